# baseline (speedup 1.0000x reference)
import functools

import jax
import jax.numpy as jnp
from jax import lax
from jax.experimental import pallas as pl
from jax.experimental.pallas import tpu as pltpu
from jax.experimental.pallas import tpu_sc as plsc

H = 64
NC, NS = 2, 16
NW = NC * NS
LC = 4
UNROLL = 4


def _rsqrt(v):
    i = lax.bitcast_convert_type(v, jnp.int32)
    i = jnp.int32(0x5F3759DF) - lax.shift_right_logical(i, 1)
    y = lax.bitcast_convert_type(i, jnp.float32)
    for _ in range(3):
        y = y * (jnp.float32(1.5) - jnp.float32(0.5) * v * y * y)
    return y


def _sc_body(wid4, fid4, wtab, ftab, gamma_in, beta_in, out,
             idx_w, idx_f, buf_w, buf_f, obuf, gam_v, bet_v, sem):
    w = lax.axis_index("s") * NC + lax.axis_index("c")
    n_lt = wid4.shape[0]
    seq = n_lt * wid4.shape[2]
    n_chunks = seq // LC

    pltpu.sync_copy(gamma_in, gam_v)
    pltpu.sync_copy(beta_in, bet_v)
    gvec = [gam_v[pl.ds(16 * k, 16)] for k in range(4)]
    bvec = [bet_v[pl.ds(16 * k, 16)] for k in range(4)]

    lanes = lax.iota(jnp.int32, 16)
    perms = [lanes ^ st for st in (8, 4, 2, 1)]

    def allsum(v):
        for p in perms:
            v = v + v.at[p].get(mode="promise_in_bounds")
        return v

    def chunk_body(c, _):
        lt = c // (8 // LC)
        lo = (c % (8 // LC)) * LC
        pltpu.sync_copy(wid4.at[lt, w, pl.ds(lo, LC)], idx_w)
        pltpu.sync_copy(fid4.at[lt, w, pl.ds(lo, LC)], idx_f)
        descs = []
        for li in range(LC):
            descs.append(pltpu.async_copy(
                wtab.at[idx_w.at[li]], buf_w.at[li], sem))
            descs.append(pltpu.async_copy(
                ftab.at[idx_f.at[li]], buf_f.at[li], sem))
        for d in descs:
            d.wait()

        def tok_body(i, _):
            for uu in range(UNROLL):
                t = i * UNROLL + uu
                for li in range(LC):
                    x = [buf_w[li, t, pl.ds(16 * k, 16)]
                         + buf_f[li, t, pl.ds(16 * k, 16)] for k in range(4)]
                    s = allsum((x[0] + x[1]) + (x[2] + x[3]))
                    q = allsum((x[0] * x[0] + x[1] * x[1])
                               + (x[2] * x[2] + x[3] * x[3]))
                    u = s * jnp.float32(1.0 / H)
                    var = q * jnp.float32(1.0 / H) - u * u
                    r = _rsqrt(var + jnp.float32(1e-12))
                    for k in range(4):
                        obuf[t, li, pl.ds(16 * k, 16)] = (
                            (x[k] - u) * r * gvec[k] + bvec[k])
            return 0

        lax.fori_loop(0, 128 // UNROLL, tok_body, 0)
        pltpu.sync_copy(obuf, out.at[pl.ds(w * 128, 128), pl.ds(c * LC, LC)])
        return 0

    lax.fori_loop(0, n_chunks, chunk_body, 0)


@functools.partial(jax.jit, static_argnums=(6, 7))
def _sc_embed(wid4, fid4, wtab, ftab, gamma, beta, batch, seq):
    mesh = plsc.VectorSubcoreMesh(core_axis_name="c", subcore_axis_name="s")
    return pl.kernel(
        _sc_body,
        out_type=jax.ShapeDtypeStruct((batch, seq, H), jnp.float32),
        mesh=mesh,
        scratch_types=[
            pltpu.VMEM((LC, 128), jnp.int32),
            pltpu.VMEM((LC, 128), jnp.int32),
            pltpu.VMEM((LC, 128, H), jnp.float32),
            pltpu.VMEM((LC, 128, H), jnp.float32),
            pltpu.VMEM((128, LC, H), jnp.float32),
            pltpu.VMEM((H,), jnp.float32),
            pltpu.VMEM((H,), jnp.float32),
            pltpu.SemaphoreType.DMA,
        ],
        compiler_params=pltpu.CompilerParams(use_tc_tiling_on_sc=False),
    )(wid4, fid4, wtab, ftab, gamma, beta)


def _tile_view(ids, B, L):
    return ids.astype(jnp.int32).reshape(B // 128, 128, L // 8, 8).transpose(
        2, 0, 3, 1)


def kernel(word_ids, age_ids, seg_ids, posi_ids, word_table, seg_table,
           age_table, posi_table, gamma, beta):
    B, L = word_ids.shape
    segv, h = seg_table.shape
    agev = age_table.shape[0]
    posv = posi_table.shape[0]
    ftab = (seg_table[:, None, None, :] + age_table[None, :, None, :]
            + posi_table[None, None, :, :]).reshape(segv * agev * posv, h)
    wid4 = _tile_view(word_ids, B, L)
    fid4 = (_tile_view(seg_ids, B, L) * agev
            + _tile_view(age_ids, B, L)) * posv + _tile_view(posi_ids, B, L)
    return _sc_embed(wid4, fid4, word_table, ftab, gamma, beta, B, L)


# speedup vs baseline: 1.9352x; 1.9352x over previous
import functools

import jax
import jax.numpy as jnp
from jax import lax
from jax.experimental import pallas as pl
from jax.experimental.pallas import tpu as pltpu
from jax.experimental.pallas import tpu_sc as plsc

H = 64
NC, NS = 2, 16
NW = NC * NS
LC = 4
UNROLL = 4


def _rsqrt(v):
    i = lax.bitcast_convert_type(v, jnp.int32)
    i = jnp.int32(0x5F3759DF) - lax.shift_right_logical(i, 1)
    y = lax.bitcast_convert_type(i, jnp.float32)
    for _ in range(3):
        y = y * (jnp.float32(1.5) - jnp.float32(0.5) * v * y * y)
    return y


def _sc_body(wid2, fid2, wtab, ftab, gamma_in, beta_in, out3,
             idx_w, idx_f, buf_w, buf_f, gam_v, bet_v, sem):
    w = lax.axis_index("s") * NC + lax.axis_index("c")
    n_rows = wid2.shape[0]
    rows_per_w = n_rows // NW
    n_chunks = rows_per_w // LC

    pltpu.sync_copy(gamma_in, gam_v)
    pltpu.sync_copy(beta_in, bet_v)
    gvec = [gam_v[pl.ds(16 * k, 16)] for k in range(4)]
    bvec = [bet_v[pl.ds(16 * k, 16)] for k in range(4)]

    lanes = lax.iota(jnp.int32, 16)
    perms = [lanes ^ st for st in (8, 4, 2, 1)]

    def allsum(v):
        for p in perms:
            v = v + v.at[p].get(mode="promise_in_bounds")
        return v

    def chunk_body(c, _):
        r0 = w * rows_per_w + c * LC
        pltpu.sync_copy(wid2.at[pl.ds(r0, LC)], idx_w)
        pltpu.sync_copy(fid2.at[pl.ds(r0, LC)], idx_f)
        descs = []
        for li in range(LC):
            descs.append(pltpu.async_copy(
                wtab.at[idx_w.at[li]], buf_w.at[li], sem))
            descs.append(pltpu.async_copy(
                ftab.at[idx_f.at[li]], buf_f.at[li], sem))
        for d in descs:
            d.wait()

        def tok_body(i, _):
            for uu in range(UNROLL):
                t = i * UNROLL + uu
                for li in range(LC):
                    x = [buf_w[li, t, pl.ds(16 * k, 16)]
                         + buf_f[li, t, pl.ds(16 * k, 16)] for k in range(4)]
                    s = allsum((x[0] + x[1]) + (x[2] + x[3]))
                    q = allsum((x[0] * x[0] + x[1] * x[1])
                               + (x[2] * x[2] + x[3] * x[3]))
                    u = s * jnp.float32(1.0 / H)
                    var = q * jnp.float32(1.0 / H) - u * u
                    r = _rsqrt(var + jnp.float32(1e-12))
                    for k in range(4):
                        buf_w[li, t, pl.ds(16 * k, 16)] = (
                            (x[k] - u) * r * gvec[k] + bvec[k])
            return 0

        lax.fori_loop(0, 128 // UNROLL, tok_body, 0)
        pltpu.sync_copy(buf_w, out3.at[pl.ds(r0, LC), :, pl.ds(0, H)])
        return 0

    lax.fori_loop(0, n_chunks, chunk_body, 0)


@functools.partial(jax.jit, static_argnums=(6,))
def _sc_embed(wid2, fid2, wtab, ftab, gamma, beta, n_rows):
    mesh = plsc.VectorSubcoreMesh(core_axis_name="c", subcore_axis_name="s")
    return pl.kernel(
        _sc_body,
        out_type=jax.ShapeDtypeStruct((n_rows, 128, 128), jnp.float32),
        mesh=mesh,
        scratch_types=[
            pltpu.VMEM((LC, 128), jnp.int32),
            pltpu.VMEM((LC, 128), jnp.int32),
            pltpu.VMEM((LC, 128, H), jnp.float32),
            pltpu.VMEM((LC, 128, H), jnp.float32),
            pltpu.VMEM((H,), jnp.float32),
            pltpu.VMEM((H,), jnp.float32),
            pltpu.SemaphoreType.DMA,
        ],
        compiler_params=pltpu.CompilerParams(use_tc_tiling_on_sc=False),
    )(wid2, fid2, wtab, ftab, gamma, beta)


def kernel(word_ids, age_ids, seg_ids, posi_ids, word_table, seg_table,
           age_table, posi_table, gamma, beta):
    B, L = word_ids.shape
    segv, h = seg_table.shape
    agev = age_table.shape[0]
    posv = posi_table.shape[0]
    n_rows = B * L // 128
    ftab = (seg_table[:, None, None, :] + age_table[None, :, None, :]
            + posi_table[None, None, :, :]).reshape(segv * agev * posv, h)
    wid2 = word_ids.astype(jnp.int32).reshape(n_rows, 128)
    fid2 = ((seg_ids.astype(jnp.int32) * agev + age_ids.astype(jnp.int32))
            * posv + posi_ids.astype(jnp.int32)).reshape(n_rows, 128)
    out3 = _sc_embed(wid2, fid2, word_table, ftab, gamma, beta, n_rows)
    return out3[:, :, :h].reshape(B, L, h)
